# baseline (speedup 1.0000x reference)
import functools

import jax
import jax.numpy as jnp
from jax import lax
from jax.experimental import pallas as pl
from jax.experimental.pallas import tpu as pltpu
from jax.experimental.pallas import tpu_sc as plsc

VOCAB = 1000000
HIDDEN = 64
SEQ = 4096
POS = 200
NUM_IDS = SEQ * POS

NC = 2
NS = 16
NW = NC * NS
SW = SEQ // NW

TB = 32768


def _detile_block(src_ref, out_ref):
    t = src_ref[...].T
    out_ref[...] = jnp.concatenate([t, t], axis=1)


_detile = pl.pallas_call(
    _detile_block,
    grid=((VOCAB + TB - 1) // TB,),
    in_specs=[pl.BlockSpec((HIDDEN, TB), lambda g: (0, g))],
    out_specs=pl.BlockSpec((TB, 128), lambda g: (g, 0)),
    out_shape=jax.ShapeDtypeStruct((VOCAB, 128), jnp.float32),
    compiler_params=pltpu.CompilerParams(vmem_limit_bytes=100 * 1024 * 1024),
)


def _make_gather():
    mesh = plsc.VectorSubcoreMesh(core_axis_name="c", subcore_axis_name="s")

    out5 = jax.ShapeDtypeStruct((POS, 8, NW, 8, 128), jnp.float32)

    @functools.partial(
        pl.kernel,
        mesh=mesh,
        out_type=(out5, out5),
        scratch_types=[
            pltpu.VMEM((POS, SW), jnp.int32),
            pltpu.VMEM((2 * SW, HIDDEN), jnp.float32),
            pltpu.VMEM((2, 8, 8, 128), jnp.float32),
            pltpu.SemaphoreType.DMA,
            pltpu.SemaphoreType.DMA,
        ],
        compiler_params=pltpu.CompilerParams(use_tc_tiling_on_sc=False, needs_layout_passes=False),
    )
    def gather_kernel(idx_hbm, table_hbm, out0_hbm, out1_hbm,
                      idx_v, rows_v, t_v, gsem, osem):
        wid = lax.axis_index("s") * NC + lax.axis_index("c")

        pltpu.sync_copy(idx_hbm.at[pl.ds(0, POS), pl.ds(wid * SW, SW)],
                        idx_v)

        lane = lax.broadcasted_iota(jnp.int32, (16,), 0)

        def fire_gather(p, slot):
            pltpu.async_copy(
                table_hbm.at[idx_v.at[p]],
                rows_v.at[pl.ds(slot * SW, SW)],
                gsem,
            )

        def wait_gather(p, slot):
            pltpu.make_async_copy(
                table_hbm.at[idx_v.at[p]],
                rows_v.at[pl.ds(slot * SW, SW)],
                gsem,
            ).wait()

        def transpose(slot):
            for h in range(HIDDEN):
                for b in range(8):
                    rows16 = lane + (slot * SW + 16 * b)
                    cols16 = jnp.full((16,), h, jnp.int32)
                    vals = plsc.load_gather(rows_v, [rows16, cols16])
                    t_v[slot, h // 8, h % 8, pl.ds(16 * b, 16)] = vals

        def fire_out(p, slot):
            pltpu.async_copy(t_v.at[slot], out0_hbm.at[p, pl.ds(0, 8), wid],
                             osem)
            pltpu.async_copy(t_v.at[slot], out1_hbm.at[p, pl.ds(0, 8), wid],
                             osem)

        def wait_out(p, slot):
            pltpu.make_async_copy(t_v.at[slot],
                                  out0_hbm.at[p, pl.ds(0, 8), wid],
                                  osem).wait()
            pltpu.make_async_copy(t_v.at[slot],
                                  out1_hbm.at[p, pl.ds(0, 8), wid],
                                  osem).wait()

        fire_gather(0, 0)

        def body(g, _):
            p0 = 2 * g
            fire_gather(p0 + 1, 1)
            wait_gather(p0, 0)

            @pl.when(g > 0)
            def _():
                wait_out(p0 - 2, 0)

            transpose(0)
            fire_out(p0, 0)

            @pl.when(g < POS // 2 - 1)
            def _():
                fire_gather(p0 + 2, 0)

            wait_gather(p0 + 1, 1)

            @pl.when(g > 0)
            def _():
                wait_out(p0 - 1, 1)

            transpose(1)
            fire_out(p0 + 1, 1)
            return 0

        lax.fori_loop(0, POS // 2, body, 0)
        wait_out(POS - 2, 0)
        wait_out(POS - 1, 1)

    return gather_kernel


_gather = _make_gather()


def kernel(input_ids, emb):
    idx = (input_ids.T * 2).astype(jnp.int32)
    table_lin = _detile(emb.T).reshape(2 * VOCAB, HIDDEN)
    h0, h1 = _gather(idx, table_lin)

    def to_canonical(x5):
        return x5.transpose(2, 4, 0, 1, 3).reshape(SEQ, POS, HIDDEN)

    return (to_canonical(h0), to_canonical(h1))


# speedup vs baseline: 2.0317x; 2.0317x over previous
import functools

import jax
import jax.numpy as jnp
from jax import lax
from jax.experimental import pallas as pl
from jax.experimental.pallas import tpu as pltpu
from jax.experimental.pallas import tpu_sc as plsc

VOCAB = 1000000
HIDDEN = 64
NUM_IDS = 4096 * 200

NC = 2
NS = 16
NW = NC * NS
B_PER_W = NUM_IDS // NW

G = 128
K = 4
CHUNK = G * K
N_CHUNKS = B_PER_W // CHUNK
NBUF = 2

TB = 32768
T_GRID = (VOCAB + TB - 1) // TB


def _detile_block(src_ref, out_ref):
    t = src_ref[...].T
    out_ref[...] = jnp.concatenate([t, t], axis=1)


_detile = pl.pallas_call(
    _detile_block,
    grid=(T_GRID,),
    in_specs=[pl.BlockSpec((HIDDEN, TB), lambda g: (0, g))],
    out_specs=pl.BlockSpec((TB, 128), lambda g: (g, 0)),
    out_shape=jax.ShapeDtypeStruct((VOCAB, 128), jnp.float32),
    compiler_params=pltpu.CompilerParams(vmem_limit_bytes=100 * 1024 * 1024),
)


def _make_gather():
    mesh = plsc.VectorSubcoreMesh(core_axis_name="c", subcore_axis_name="s")

    @functools.partial(
        pl.kernel,
        mesh=mesh,
        out_type=jax.ShapeDtypeStruct((NUM_IDS, 128), jnp.float32),
        scratch_types=[
            pltpu.VMEM((B_PER_W,), jnp.int32),
            pltpu.VMEM((NBUF * CHUNK, HIDDEN), jnp.float32),
            pltpu.SemaphoreType.DMA,
        ],
        compiler_params=pltpu.CompilerParams(use_tc_tiling_on_sc=False),
    )
    def gather_kernel(idx_hbm, table_hbm, out_hbm, idx_v, rows_v, gsem):
        wid = lax.axis_index("s") * NC + lax.axis_index("c")
        base = wid * B_PER_W

        pltpu.sync_copy(idx_hbm.at[pl.ds(base, B_PER_W)], idx_v)

        def fire(i, slot):
            voff = slot * CHUNK
            for j in range(K):
                pltpu.async_copy(
                    table_hbm.at[idx_v.at[pl.ds(i * CHUNK + j * G, G)]],
                    rows_v.at[pl.ds(voff + j * G, G)],
                    gsem,
                )

        def drain_and_store(i, slot):
            off = base + i * CHUNK
            voff = slot * CHUNK
            for j in range(K):
                pltpu.make_async_copy(
                    table_hbm.at[idx_v.at[pl.ds(i * CHUNK + j * G, G)]],
                    rows_v.at[pl.ds(voff + j * G, G)],
                    gsem,
                ).wait()
            pltpu.sync_copy(rows_v.at[pl.ds(voff, CHUNK)],
                            out_hbm.at[pl.ds(off, CHUNK), pl.ds(0, HIDDEN)])

        fire(0, 0)

        def body(i, _):
            @pl.when(i + 1 < N_CHUNKS)
            def _():
                fire(i + 1, lax.rem(i + 1, NBUF))

            drain_and_store(i, lax.rem(i, NBUF))
            return 0

        lax.fori_loop(0, N_CHUNKS, body, 0)

    return gather_kernel


_gather = _make_gather()


def kernel(input_ids, emb):
    idx = input_ids.reshape(-1).astype(jnp.int32) * 2
    table_lin = _detile(emb.T).reshape(2 * VOCAB, HIDDEN)
    h = _gather(idx, table_lin)
    h = h[:, :HIDDEN].reshape(input_ids.shape + (HIDDEN,))
    return (h, h)
